# baseline (speedup 1.0000x reference)
import functools

import jax
import jax.numpy as jnp
from jax import lax
from jax.experimental import pallas as pl
from jax.experimental.pallas import tpu as pltpu
from jax.experimental.pallas import tpu_sc as plsc

B = 16384
EG = 16
EM = 32
NF = EG + EM + EM
DOUT = EG + EM + EM
NC = 2
NS = 16
NW = NC * NS
BPW = B // NW
CHUNK = 128
NCH = BPW // CHUNK


@functools.cache
def _get_sc_gather():
    mesh = plsc.VectorSubcoreMesh(core_axis_name="c", subcore_axis_name="s")

    @functools.partial(
        pl.kernel,
        mesh=mesh,
        out_type=jax.ShapeDtypeStruct((B * DOUT,), jnp.float32),
        scratch_types=[
            pltpu.VMEM((NCH, CHUNK), jnp.int32),
            pltpu.VMEM((NCH, CHUNK), jnp.int32),
            pltpu.VMEM((2 * EG + 2 * EM, BPW), jnp.float32),
            pltpu.VMEM((BPW * DOUT,), jnp.float32),
            pltpu.SemaphoreType.DMA,
        ],
        compiler_params=pltpu.CompilerParams(
            use_tc_tiling_on_sc=False, needs_layout_passes=False),
    )
    def _sc_gather(u_hbm, i_hbm, ugt, igt, umt, imt, out,
                   idx_u, idx_i, gath, st, sem):
        wid = lax.axis_index("s") * NC + lax.axis_index("c")
        pltpu.sync_copy(u_hbm.at[pl.ds(wid * NCH, NCH)], idx_u)
        pltpu.sync_copy(i_hbm.at[pl.ds(wid * NCH, NCH)], idx_i)
        cps = []
        for f in range(EG):
            for c in range(NCH):
                sl = pl.ds(c * CHUNK, CHUNK)
                cps.append(pltpu.async_copy(
                    ugt.at[f].at[idx_u.at[c]], gath.at[f, sl], sem))
                cps.append(pltpu.async_copy(
                    igt.at[f].at[idx_i.at[c]], gath.at[EG + f, sl], sem))
        for f in range(EM):
            for c in range(NCH):
                sl = pl.ds(c * CHUNK, CHUNK)
                cps.append(pltpu.async_copy(
                    umt.at[f].at[idx_u.at[c]], gath.at[2 * EG + f, sl], sem))
                cps.append(pltpu.async_copy(
                    imt.at[f].at[idx_i.at[c]], gath.at[2 * EG + EM + f, sl], sem))
        for cp in cps:
            cp.wait()

        lanes = lax.iota(jnp.int32, 16)

        def body(r, _):
            rbase = (r * 16) * DOUT
            tgt = lanes * DOUT + rbase
            sl = pl.ds(r * 16, 16)
            for f in range(EG):
                v = gath[f, sl] * gath[EG + f, sl]
                plsc.store_scatter(st, [tgt + f], v)
            for f in range(EM):
                v = gath[2 * EG + f, sl]
                plsc.store_scatter(st, [tgt + (EG + f)], v)
                v = gath[2 * EG + EM + f, sl]
                plsc.store_scatter(st, [tgt + (EG + EM + f)], v)
            return _

        lax.fori_loop(0, BPW // 16, body, None)
        pltpu.sync_copy(st, out.at[pl.ds(wid * (BPW * DOUT), BPW * DOUT)])

    return _sc_gather


BLK = 4096


def _mlp_body(x, w1u, w1i, b1, w2, b2, w3, b3, wfg, wfm, bfr, out):
    xx = x[...]
    g = xx[:, :EG]
    um = xx[:, EG:EG + EM]
    im = xx[:, EG + EM:]
    h = jnp.dot(um, w1u[...], preferred_element_type=jnp.float32)
    h = h + jnp.dot(im, w1i[...], preferred_element_type=jnp.float32)
    h = jnp.maximum(h + b1[...], 0.0)
    h = jnp.maximum(jnp.dot(h, w2[...], preferred_element_type=jnp.float32) + b2[...], 0.0)
    h = jnp.maximum(jnp.dot(h, w3[...], preferred_element_type=jnp.float32) + b3[...], 0.0)
    z = jnp.sum(g * wfg[...], axis=1) + jnp.sum(h * wfm[...], axis=1)
    out[...] = jax.nn.sigmoid(z + jnp.sum(bfr[...]))


def _full(shape):
    return pl.BlockSpec(shape, lambda b: (0,) * len(shape))


def kernel(u, i, user_gmf, item_gmf, user_mlp, item_mlp,
           W1, b1, W2, b2, W3, b3, Wf, bf):
    u2 = u.astype(jnp.int32).reshape(B // CHUNK, CHUNK)
    i2 = i.astype(jnp.int32).reshape(B // CHUNK, CHUNK)
    x = _get_sc_gather()(u2, i2, user_gmf.T, item_gmf.T, user_mlp.T, item_mlp.T)
    x = x.reshape(B, DOUT)

    w1u, w1i = W1[:EM, :], W1[EM:, :]
    wfg, wfm = Wf[:EG, 0].reshape(1, EG), Wf[EG:, 0].reshape(1, EG)
    grid = B // BLK
    out = pl.pallas_call(
        _mlp_body,
        grid=(grid,),
        in_specs=[
            pl.BlockSpec((BLK, DOUT), lambda b: (b, 0)),
            _full((EM, 64)), _full((EM, 64)), _full((1, 64)),
            _full((64, 32)), _full((1, 32)),
            _full((32, EG)), _full((1, EG)),
            _full((1, EG)), _full((1, EG)), _full((1, 1)),
        ],
        out_specs=pl.BlockSpec((BLK,), lambda b: (b,)),
        out_shape=jax.ShapeDtypeStruct((B,), jnp.float32),
    )(x, w1u, w1i, b1.reshape(1, 64), W2, b2.reshape(1, 32),
      W3, b3.reshape(1, EG), wfg, wfm, bf.reshape(1, 1))
    return out


# speedup vs baseline: 19.2678x; 19.2678x over previous
import functools

import jax
import jax.numpy as jnp
from jax import lax
from jax.experimental import pallas as pl
from jax.experimental.pallas import tpu as pltpu
from jax.experimental.pallas import tpu_sc as plsc

B = 16384
EG = 16
EM = 32
DOUT = EG + EM + EM
NC = 2
NS = 16
NW = NC * NS
BPW = B // NW
CH = 2
NCHK = BPW // CH
LW = 128
ROWS_PER_IDX = 2 * (EG + EM)


@functools.cache
def _get_sc_gather():
    mesh = plsc.VectorSubcoreMesh(core_axis_name="c", subcore_axis_name="s")

    @functools.partial(
        pl.kernel,
        mesh=mesh,
        out_type=jax.ShapeDtypeStruct((B * DOUT,), jnp.float32),
        scratch_types=[
            pltpu.SMEM((BPW,), jnp.int32),
            pltpu.SMEM((BPW,), jnp.int32),
            pltpu.VMEM((CH * ROWS_PER_IDX, LW), jnp.float32),
            pltpu.VMEM((CH * ROWS_PER_IDX, LW), jnp.float32),
            pltpu.VMEM((BPW,), jnp.int32),
            pltpu.VMEM((BPW,), jnp.int32),
            pltpu.VMEM((BPW * DOUT,), jnp.float32),
            pltpu.SemaphoreType.DMA,
            pltpu.SemaphoreType.DMA,
        ],
        compiler_params=pltpu.CompilerParams(needs_layout_passes=False),
    )
    def _sc_gather(u_hbm, i_hbm, ugt, igt, umt, imt, out,
                   su, si, b0, b1, ivu, ivi, st, sem0, sem1):
        wid = lax.axis_index("s") * NC + lax.axis_index("c")
        base = wid * BPW
        pltpu.sync_copy(u_hbm.at[pl.ds(base, BPW)], ivu)
        pltpu.sync_copy(i_hbm.at[pl.ds(base, BPW)], ivi)

        def _fill(g, _):
            vu = ivu[pl.ds(g * 16, 16)]
            vi = ivi[pl.ds(g * 16, 16)]
            for l in range(16):
                su[g * 16 + l] = vu[l]
                si[g * 16 + l] = vi[l]
            return _

        lax.fori_loop(0, BPW // 16, _fill, None)
        lanes = lax.iota(jnp.int32, 16)

        def issue(chunk, bb, sem):
            def body(j, _):
                rbase = j * ROWS_PER_IDX
                ui = su[chunk * CH + j]
                ii = si[chunk * CH + j]
                uc = pl.multiple_of((ui >> 7) << 7, LW)
                ic = pl.multiple_of((ii >> 7) << 7, LW)
                pltpu.async_copy(ugt.at[:, pl.ds(uc, LW)],
                                 bb.at[pl.ds(rbase, EG), :], sem)
                pltpu.async_copy(umt.at[:, pl.ds(uc, LW)],
                                 bb.at[pl.ds(rbase + EG, EM), :], sem)
                pltpu.async_copy(igt.at[:, pl.ds(ic, LW)],
                                 bb.at[pl.ds(rbase + EG + EM, EG), :], sem)
                pltpu.async_copy(imt.at[:, pl.ds(ic, LW)],
                                 bb.at[pl.ds(rbase + 2 * EG + EM, EM), :], sem)
                return _

            lax.fori_loop(0, CH, body, None)

        def drain(bb, sem):
            def body(j, _):
                rbase = j * ROWS_PER_IDX
                pltpu.make_async_copy(ugt.at[:, pl.ds(0, LW)],
                                      bb.at[pl.ds(rbase, EG), :], sem).wait()
                pltpu.make_async_copy(umt.at[:, pl.ds(0, LW)],
                                      bb.at[pl.ds(rbase + EG, EM), :], sem).wait()
                pltpu.make_async_copy(igt.at[:, pl.ds(0, LW)],
                                      bb.at[pl.ds(rbase + EG + EM, EG), :], sem).wait()
                pltpu.make_async_copy(imt.at[:, pl.ds(0, LW)],
                                      bb.at[pl.ds(rbase + 2 * EG + EM, EM), :], sem).wait()
                return _

            lax.fori_loop(0, CH, body, None)

        def extract(chunk, bb):
            def body(j, _):
                rbase = j * ROWS_PER_IDX
                ul = jnp.broadcast_to(su[chunk * CH + j] & (LW - 1), (16,))
                il = jnp.broadcast_to(si[chunk * CH + j] & (LW - 1), (16,))
                ug = plsc.load_gather(bb, [rbase + lanes, ul])
                ig = plsc.load_gather(bb, [rbase + EG + EM + lanes, il])
                um0 = plsc.load_gather(bb, [rbase + EG + lanes, ul])
                um1 = plsc.load_gather(bb, [rbase + EG + 16 + lanes, ul])
                im0 = plsc.load_gather(bb, [rbase + 2 * EG + EM + lanes, il])
                im1 = plsc.load_gather(bb, [rbase + 2 * EG + EM + 16 + lanes, il])
                sbase = (chunk * CH + j) * DOUT
                st[pl.ds(sbase, 16)] = ug * ig
                st[pl.ds(sbase + EG, 16)] = um0
                st[pl.ds(sbase + EG + 16, 16)] = um1
                st[pl.ds(sbase + EG + EM, 16)] = im0
                st[pl.ds(sbase + EG + EM + 16, 16)] = im1
                return _

            lax.fori_loop(0, CH, body, None)

        issue(0, b0, sem0)

        def step(c, cur, nxt, scur, snxt):
            @pl.when(c + 1 < NCHK)
            def _issue_next():
                issue(c + 1, nxt, snxt)

            drain(cur, scur)
            extract(c, cur)

        def chunk_body(c, _):
            @pl.when(c % 2 == 0)
            def _even():
                step(c, b0, b1, sem0, sem1)

            @pl.when(c % 2 == 1)
            def _odd():
                step(c, b1, b0, sem1, sem0)

            return _

        lax.fori_loop(0, NCHK, chunk_body, None)
        pltpu.sync_copy(st, out.at[pl.ds(wid * (BPW * DOUT), BPW * DOUT)])

    return _sc_gather


BLK = 4096


def _mlp_body(x, w1u, w1i, b1, w2, b2, w3, b3, wfg, wfm, bfr, out):
    xx = x[...]
    g = xx[:, :EG]
    um = xx[:, EG:EG + EM]
    im = xx[:, EG + EM:]
    h = jnp.dot(um, w1u[...], preferred_element_type=jnp.float32)
    h = h + jnp.dot(im, w1i[...], preferred_element_type=jnp.float32)
    h = jnp.maximum(h + b1[...], 0.0)
    h = jnp.maximum(jnp.dot(h, w2[...], preferred_element_type=jnp.float32) + b2[...], 0.0)
    h = jnp.maximum(jnp.dot(h, w3[...], preferred_element_type=jnp.float32) + b3[...], 0.0)
    z = jnp.sum(g * wfg[...], axis=1) + jnp.sum(h * wfm[...], axis=1)
    out[...] = jax.nn.sigmoid(z + jnp.sum(bfr[...]))


def _full(shape):
    return pl.BlockSpec(shape, lambda b: (0,) * len(shape))


def kernel(u, i, user_gmf, item_gmf, user_mlp, item_mlp,
           W1, b1, W2, b2, W3, b3, Wf, bf):
    x = _get_sc_gather()(u.astype(jnp.int32), i.astype(jnp.int32),
                         user_gmf.T, item_gmf.T, user_mlp.T, item_mlp.T)
    x = x.reshape(B, DOUT)

    w1u, w1i = W1[:EM, :], W1[EM:, :]
    wfg, wfm = Wf[:EG, 0].reshape(1, EG), Wf[EG:, 0].reshape(1, EG)
    grid = B // BLK
    out = pl.pallas_call(
        _mlp_body,
        grid=(grid,),
        in_specs=[
            pl.BlockSpec((BLK, DOUT), lambda b: (b, 0)),
            _full((EM, 64)), _full((EM, 64)), _full((1, 64)),
            _full((64, 32)), _full((1, 32)),
            _full((32, EG)), _full((1, EG)),
            _full((1, EG)), _full((1, EG)), _full((1, 1)),
        ],
        out_specs=pl.BlockSpec((BLK,), lambda b: (b,)),
        out_shape=jax.ShapeDtypeStruct((B,), jnp.float32),
    )(x, w1u, w1i, b1.reshape(1, 64), W2, b2.reshape(1, 32),
      W3, b3.reshape(1, EG), wfg, wfm, bf.reshape(1, 1))
    return out


# speedup vs baseline: 20.4484x; 1.0613x over previous
import functools

import jax
import jax.numpy as jnp
from jax import lax
from jax.experimental import pallas as pl
from jax.experimental.pallas import tpu as pltpu
from jax.experimental.pallas import tpu_sc as plsc

B = 16384
EG = 16
EM = 32
DOUT = EG + EM + EM
NC = 2
NS = 16
NW = NC * NS
BPW = B // NW
CH = 4
NCHK = BPW // CH
LW = 128
ROWS_PER_IDX = 2 * (EG + EM)


@functools.cache
def _get_sc_gather():
    mesh = plsc.VectorSubcoreMesh(core_axis_name="c", subcore_axis_name="s")

    @functools.partial(
        pl.kernel,
        mesh=mesh,
        out_type=jax.ShapeDtypeStruct((B * DOUT,), jnp.float32),
        scratch_types=[
            pltpu.SMEM((BPW,), jnp.int32),
            pltpu.SMEM((BPW,), jnp.int32),
            pltpu.VMEM((CH * ROWS_PER_IDX, LW), jnp.float32),
            pltpu.VMEM((CH * ROWS_PER_IDX, LW), jnp.float32),
            pltpu.VMEM((BPW,), jnp.int32),
            pltpu.VMEM((BPW,), jnp.int32),
            pltpu.VMEM((CH * DOUT,), jnp.float32),
            pltpu.VMEM((CH * DOUT,), jnp.float32),
            pltpu.SemaphoreType.DMA,
            pltpu.SemaphoreType.DMA,
            pltpu.SemaphoreType.DMA,
            pltpu.SemaphoreType.DMA,
        ],
        compiler_params=pltpu.CompilerParams(needs_layout_passes=False),
    )
    def _sc_gather(u_hbm, i_hbm, ugt, igt, umt, imt, dumm, out,
                   su, si, b0, b1, ivu, ivi, st0, st1, sem0, sem1, semo0, semo1):
        wid = lax.axis_index("s") * NC + lax.axis_index("c")
        base = wid * BPW
        pltpu.sync_copy(u_hbm.at[pl.ds(base, BPW)], ivu)
        pltpu.sync_copy(i_hbm.at[pl.ds(base, BPW)], ivi)

        def _fill(g, _):
            vu = ivu[pl.ds(g * 16, 16)]
            vi = ivi[pl.ds(g * 16, 16)]
            for l in range(16):
                su[g * 16 + l] = vu[l]
                si[g * 16 + l] = vi[l]
            return _

        lax.fori_loop(0, BPW // 16, _fill, None)
        lanes = lax.iota(jnp.int32, 16)

        def issue(chunk, bb, sem):
            def body(j, _):
                rbase = j * ROWS_PER_IDX
                ui = su[chunk * CH + j]
                ii = si[chunk * CH + j]
                uc = pl.multiple_of((ui >> 7) << 7, LW)
                ic = pl.multiple_of((ii >> 7) << 7, LW)
                pltpu.async_copy(ugt.at[:, pl.ds(uc, LW)],
                                 bb.at[pl.ds(rbase, EG), :], sem)
                pltpu.async_copy(umt.at[:, pl.ds(uc, LW)],
                                 bb.at[pl.ds(rbase + EG, EM), :], sem)
                pltpu.async_copy(igt.at[:, pl.ds(ic, LW)],
                                 bb.at[pl.ds(rbase + EG + EM, EG), :], sem)
                pltpu.async_copy(imt.at[:, pl.ds(ic, LW)],
                                 bb.at[pl.ds(rbase + 2 * EG + EM, EM), :], sem)
                return _

            lax.fori_loop(0, CH, body, None)

        def drain(bb, sem):
            pltpu.make_async_copy(dumm, bb, sem).wait()

        def extract(chunk, bb, st):
            def body(j, _):
                rbase = j * ROWS_PER_IDX
                ul = jnp.broadcast_to(su[chunk * CH + j] & (LW - 1), (16,))
                il = jnp.broadcast_to(si[chunk * CH + j] & (LW - 1), (16,))
                ug = plsc.load_gather(bb, [rbase + lanes, ul])
                ig = plsc.load_gather(bb, [rbase + EG + EM + lanes, il])
                um0 = plsc.load_gather(bb, [rbase + EG + lanes, ul])
                um1 = plsc.load_gather(bb, [rbase + EG + 16 + lanes, ul])
                im0 = plsc.load_gather(bb, [rbase + 2 * EG + EM + lanes, il])
                im1 = plsc.load_gather(bb, [rbase + 2 * EG + EM + 16 + lanes, il])
                sbase = j * DOUT
                st[pl.ds(sbase, 16)] = ug * ig
                st[pl.ds(sbase + EG, 16)] = um0
                st[pl.ds(sbase + EG + 16, 16)] = um1
                st[pl.ds(sbase + EG + EM, 16)] = im0
                st[pl.ds(sbase + EG + EM + 16, 16)] = im1
                return _

            lax.fori_loop(0, CH, body, None)

        issue(0, b0, sem0)

        def step(c, cur, nxt, scur, snxt, st, semo):
            @pl.when(c + 1 < NCHK)
            def _issue_next():
                issue(c + 1, nxt, snxt)

            drain(cur, scur)

            @pl.when(c >= 2)
            def _reclaim_st():
                pltpu.make_async_copy(
                    out.at[pl.ds(0, CH * DOUT)], st, semo).wait()

            extract(c, cur, st)
            pltpu.async_copy(
                st, out.at[pl.ds((base + c * CH) * DOUT, CH * DOUT)], semo)

        def chunk_body(c, _):
            @pl.when(c % 2 == 0)
            def _even():
                step(c, b0, b1, sem0, sem1, st0, semo0)

            @pl.when(c % 2 == 1)
            def _odd():
                step(c, b1, b0, sem1, sem0, st1, semo1)

            return _

        lax.fori_loop(0, NCHK, chunk_body, None)
        pltpu.make_async_copy(out.at[pl.ds(0, CH * DOUT)], st0, semo0).wait()
        pltpu.make_async_copy(out.at[pl.ds(0, CH * DOUT)], st1, semo1).wait()

    return _sc_gather


BLK = 4096


def _mlp_body(x, w1u, w1i, b1, w2, b2, w3, b3, wfg, wfm, bfr, out):
    xx = x[...]
    g = xx[:, :EG]
    um = xx[:, EG:EG + EM]
    im = xx[:, EG + EM:]
    h = jnp.dot(um, w1u[...], preferred_element_type=jnp.float32)
    h = h + jnp.dot(im, w1i[...], preferred_element_type=jnp.float32)
    h = jnp.maximum(h + b1[...], 0.0)
    h = jnp.maximum(jnp.dot(h, w2[...], preferred_element_type=jnp.float32) + b2[...], 0.0)
    h = jnp.maximum(jnp.dot(h, w3[...], preferred_element_type=jnp.float32) + b3[...], 0.0)
    z = jnp.sum(g * wfg[...], axis=1) + jnp.sum(h * wfm[...], axis=1)
    out[...] = jax.nn.sigmoid(z + jnp.sum(bfr[...]))


def _full(shape):
    return pl.BlockSpec(shape, lambda b: (0,) * len(shape))


def kernel(u, i, user_gmf, item_gmf, user_mlp, item_mlp,
           W1, b1, W2, b2, W3, b3, Wf, bf):
    dumm = jnp.zeros((CH * ROWS_PER_IDX, LW), jnp.float32)
    x = _get_sc_gather()(u.astype(jnp.int32), i.astype(jnp.int32),
                         user_gmf.T, item_gmf.T, user_mlp.T, item_mlp.T, dumm)
    x = x.reshape(B, DOUT)

    w1u, w1i = W1[:EM, :], W1[EM:, :]
    wfg, wfm = Wf[:EG, 0].reshape(1, EG), Wf[EG:, 0].reshape(1, EG)
    grid = B // BLK
    out = pl.pallas_call(
        _mlp_body,
        grid=(grid,),
        in_specs=[
            pl.BlockSpec((BLK, DOUT), lambda b: (b, 0)),
            _full((EM, 64)), _full((EM, 64)), _full((1, 64)),
            _full((64, 32)), _full((1, 32)),
            _full((32, EG)), _full((1, EG)),
            _full((1, EG)), _full((1, EG)), _full((1, 1)),
        ],
        out_specs=pl.BlockSpec((BLK,), lambda b: (b,)),
        out_shape=jax.ShapeDtypeStruct((B,), jnp.float32),
    )(x, w1u, w1i, b1.reshape(1, 64), W2, b2.reshape(1, 32),
      W3, b3.reshape(1, EG), wfg, wfm, bf.reshape(1, 1))
    return out
